# baseline (speedup 1.0000x reference)
import jax
import jax.numpy as jnp
from jax import lax
from jax.experimental import pallas as pl
from jax.experimental.pallas import tpu as pltpu
from jax.experimental.pallas import tpu_sc as plsc

L = 16
NB = 256
ROWS = 128
N = 32768
VECS = N // L
NWORKERS = 32
ROWS_PER = ROWS // NWORKERS
W1 = 1.0 / NB
W2 = W1 / NB
U = 8


def _splat(s, dtype=None):
    v = lax.broadcast(s, (L,))
    return v if dtype is None else v.astype(dtype)


def _sparsemax_body(in_hbm, out_hbm, row_v, hcnt, hsum):
    c = lax.axis_index("c")
    s = lax.axis_index("s")
    wid = s * 2 + c

    lane = lax.iota(jnp.int32, L)
    ones = jnp.ones((L,), jnp.float32)
    zeros = jnp.zeros((L,), jnp.float32)

    def hist_level(top_vec, bottom_vec, inv_w, w):
        def zero_body(b, carry):
            for j in range(U):
                hcnt[b * U + j] = zeros
                hsum[b * U + j] = zeros
            return carry
        lax.fori_loop(0, NB // U, zero_body, 0)
        hcnt[NB] = zeros
        hsum[NB] = zeros

        inv_w_vec = jnp.full((L,), inv_w, jnp.float32)

        def scat_body(i, carry):
            for j in range(U):
                x = row_v[pl.ds(pl.multiple_of((i * U + j) * L, L), L)]
                mask = x > bottom_vec
                tt = (top_vec - x) * inv_w_vec
                idx = jnp.clip(tt.astype(jnp.int32), 0, NB)
                plsc.addupdate_scatter(hcnt, [idx, lane], ones, mask=mask)
                plsc.addupdate_scatter(hsum, [idx, lane], x, mask=mask)
            return carry
        lax.fori_loop(0, VECS // U, scat_body, 0)

        def cum_body(b, carry):
            cc, cs = carry
            for j in range(U):
                cc = cc + hcnt[b * U + j]
                cs = cs + hsum[b * U + j]
                hcnt[b * U + j] = cc
                hsum[b * U + j] = cs
            return (cc, cs)
        lax.fori_loop(0, NB // U, cum_body, (zeros, zeros))

        w_vec = jnp.full((L,), w, jnp.float32)

        def g_nonneg(b):
            cvec = _splat(jnp.sum(hcnt[b]))
            svec = _splat(jnp.sum(hsum[b]))
            bf = _splat(b + 1).astype(jnp.float32)
            beta = top_vec - bf * w_vec
            g = svec - beta * cvec - ones
            return jnp.any(g >= 0.0)

        def bs_body(it, lohi):
            lo, hi = lohi
            mid = (lo + hi) >> 1
            pred = g_nonneg(mid)
            lo2 = jnp.where(pred, lo, mid + 1)
            hi2 = jnp.where(pred, mid, hi)
            done = lo >= hi
            return (jnp.where(done, lo, lo2), jnp.where(done, hi, hi2))

        bstar, _ = lax.fori_loop(0, 8, bs_body,
                                 (jnp.int32(0), jnp.int32(NB - 1)))
        kvec = _splat(jnp.sum(hcnt[bstar]))
        svec = _splat(jnp.sum(hsum[bstar]))
        bf = _splat(bstar).astype(jnp.float32)
        new_top = top_vec - bf * w_vec
        return kvec, svec, new_top

    def do_row(r, carry):
        row = wid * ROWS_PER + r
        pltpu.sync_copy(in_hbm.at[row], row_v)

        def max_body(i, accs):
            return tuple(
                jnp.maximum(a, row_v[pl.ds(pl.multiple_of((i * U + j) * L, L),
                                           L)])
                for j, a in enumerate(accs))
        accs = lax.fori_loop(
            0, VECS // U, max_body,
            tuple(jnp.full((L,), -jnp.inf, jnp.float32) for _ in range(U)))
        acc = accs[0]
        for j in range(1, U):
            acc = jnp.maximum(acc, accs[j])
        m_vec = _splat(jnp.max(acc))

        _, _, top2 = hist_level(m_vec, m_vec - ones, float(NB), W1)
        w1_vec = jnp.full((L,), W1, jnp.float32)
        kvec, svec, _ = hist_level(top2, top2 - w1_vec, float(NB * NB), W2)
        tau = (svec - ones) / kvec

        def out_body(i, carry):
            for j in range(U):
                sl = pl.ds(pl.multiple_of((i * U + j) * L, L), L)
                row_v[sl] = jnp.maximum(row_v[sl] - tau, 0.0)
            return carry
        lax.fori_loop(0, VECS // U, out_body, 0)
        pltpu.sync_copy(row_v, out_hbm.at[row])
        return carry

    lax.fori_loop(0, ROWS_PER, do_row, 0)


@jax.jit
def _sparsemax_sc(input_):
    mesh = plsc.VectorSubcoreMesh(core_axis_name="c", subcore_axis_name="s",
                                  num_cores=2, num_subcores=16)
    f = pl.kernel(
        _sparsemax_body,
        out_type=jax.ShapeDtypeStruct((ROWS, N), jnp.float32),
        mesh=mesh,
        scratch_types=[
            pltpu.VMEM((N,), jnp.float32),
            pltpu.VMEM((NB + 1, L), jnp.float32),
            pltpu.VMEM((NB + 1, L), jnp.float32),
        ],
        compiler_params=pltpu.CompilerParams(
            needs_layout_passes=False, use_tc_tiling_on_sc=False),
    )
    return f(input_)


def kernel(input_):
    return _sparsemax_sc(input_)


# speedup vs baseline: 1.0687x; 1.0687x over previous
import jax
import jax.numpy as jnp
from jax import lax
from jax.experimental import pallas as pl
from jax.experimental.pallas import tpu as pltpu
from jax.experimental.pallas import tpu_sc as plsc

L = 16
NB = 256
ROWS = 128
N = 32768
VECS = N // L
NWORKERS = 32
ROWS_PER = ROWS // NWORKERS
W1 = 1.0 / NB
W2 = W1 / NB
U = 8
CAP = 4096


def _splat(s, dtype=None):
    v = lax.broadcast(s, (L,))
    return v if dtype is None else v.astype(dtype)


def _sparsemax_body(in_hbm, out_hbm, row_v, cand_x, hcnt, hsum):
    c = lax.axis_index("c")
    s = lax.axis_index("s")
    wid = s * 2 + c

    lane = lax.iota(jnp.int32, L)
    ones = jnp.ones((L,), jnp.float32)
    zeros = jnp.zeros((L,), jnp.float32)
    cap_vec = jnp.full((L,), CAP, jnp.int32)

    def hist_level(top_vec, inv_w, w, use_cand, nv, tail):
        def zero_body(b, carry):
            for j in range(U):
                hcnt[b * U + j] = zeros
                hsum[b * U + j] = zeros
            return carry
        lax.fori_loop(0, NB // U, zero_body, 0)
        hcnt[NB] = zeros
        hsum[NB] = zeros

        inv_w_vec = jnp.full((L,), inv_w, jnp.float32)

        def scat_one(x, mask=None):
            tt = (top_vec - x) * inv_w_vec
            idx = jnp.clip(tt.astype(jnp.int32), 0, NB)
            plsc.addupdate_scatter(hcnt, [idx, lane], ones, mask=mask)
            plsc.addupdate_scatter(hsum, [idx, lane], x, mask=mask)

        @pl.when(use_cand)
        def _():
            def body(i, carry):
                scat_one(cand_x[pl.ds(pl.multiple_of(i * L, L), L)])
                return carry
            lax.fori_loop(0, nv, body, 0)
            xt = cand_x[pl.ds(pl.multiple_of(nv * L, L), L)]
            scat_one(xt, mask=lane < _splat(tail))

        @pl.when(jnp.logical_not(use_cand))
        def _():
            def body(i, carry):
                for j in range(U):
                    scat_one(row_v[pl.ds(pl.multiple_of((i * U + j) * L, L),
                                         L)])
                return carry
            lax.fori_loop(0, VECS // U, body, 0)

        def cum_body(b, carry):
            cc, cs = carry
            for j in range(U):
                cc = cc + hcnt[b * U + j]
                cs = cs + hsum[b * U + j]
                hcnt[b * U + j] = cc
                hsum[b * U + j] = cs
            return (cc, cs)
        lax.fori_loop(0, NB // U, cum_body, (zeros, zeros))

        w_vec = jnp.full((L,), w, jnp.float32)

        def g_nonneg(b):
            cvec = _splat(jnp.sum(hcnt[b]))
            svec = _splat(jnp.sum(hsum[b]))
            bf = _splat(b + 1).astype(jnp.float32)
            beta = top_vec - bf * w_vec
            g = svec - beta * cvec - ones
            return jnp.any(g >= 0.0)

        def bs_body(it, lohi):
            lo, hi = lohi
            mid = (lo + hi) >> 1
            pred = g_nonneg(mid)
            lo2 = jnp.where(pred, lo, mid + 1)
            hi2 = jnp.where(pred, mid, hi)
            done = lo >= hi
            return (jnp.where(done, lo, lo2), jnp.where(done, hi, hi2))

        bstar, _ = lax.fori_loop(0, 8, bs_body,
                                 (jnp.int32(0), jnp.int32(NB - 1)))
        kvec = _splat(jnp.sum(hcnt[bstar]))
        svec = _splat(jnp.sum(hsum[bstar]))
        bf = _splat(bstar).astype(jnp.float32)
        new_top = top_vec - bf * w_vec
        return kvec, svec, new_top

    def do_row(r, carry):
        row = wid * ROWS_PER + r
        pltpu.sync_copy(in_hbm.at[row], row_v)

        def max_body(i, accs):
            return tuple(
                jnp.maximum(a, row_v[pl.ds(pl.multiple_of((i * U + j) * L, L),
                                           L)])
                for j, a in enumerate(accs))
        accs = lax.fori_loop(
            0, VECS // U, max_body,
            tuple(jnp.full((L,), -jnp.inf, jnp.float32) for _ in range(U)))
        acc = accs[0]
        for j in range(1, U):
            acc = jnp.maximum(acc, accs[j])
        m_vec = _splat(jnp.max(acc))

        thresh = m_vec - ones

        def comp_body(i, cnt_vec):
            for j in range(U):
                x = row_v[pl.ds(pl.multiple_of((i * U + j) * L, L), L)]
                mask = x > thresh
                pc = plsc.all_reduce_population_count(mask)
                cur = cnt_vec

                @pl.when(pc[0] > 0)
                def _():
                    pref = plsc.cumsum(mask.astype(jnp.int32))
                    dest = cur + pref - 1
                    okm = jnp.logical_and(mask, dest < cap_vec)
                    plsc.store_scatter(cand_x, [dest], x, mask=okm)

                cnt_vec = cnt_vec + pc
            return cnt_vec
        ncand_vec = lax.fori_loop(0, VECS // U, comp_body,
                                  jnp.zeros((L,), jnp.int32))
        ncand = ncand_vec[0]

        use_cand = ncand <= CAP
        nv = lax.shift_right_logical(ncand, 4)
        tail = jnp.bitwise_and(ncand, 15)

        _, _, top2 = hist_level(m_vec, float(NB), W1, use_cand, nv, tail)
        kvec, svec, _ = hist_level(top2, float(NB * NB), W2,
                                   use_cand, nv, tail)
        tau = (svec - ones) / kvec

        def out_body(i, carry):
            for j in range(U):
                sl = pl.ds(pl.multiple_of((i * U + j) * L, L), L)
                row_v[sl] = jnp.maximum(row_v[sl] - tau, 0.0)
            return carry
        lax.fori_loop(0, VECS // U, out_body, 0)
        pltpu.sync_copy(row_v, out_hbm.at[row])
        return carry

    lax.fori_loop(0, ROWS_PER, do_row, 0)


@jax.jit
def _sparsemax_sc(input_):
    mesh = plsc.VectorSubcoreMesh(core_axis_name="c", subcore_axis_name="s",
                                  num_cores=2, num_subcores=16)
    f = pl.kernel(
        _sparsemax_body,
        out_type=jax.ShapeDtypeStruct((ROWS, N), jnp.float32),
        mesh=mesh,
        scratch_types=[
            pltpu.VMEM((N,), jnp.float32),
            pltpu.VMEM((CAP + L,), jnp.float32),
            pltpu.VMEM((NB + 1, L), jnp.float32),
            pltpu.VMEM((NB + 1, L), jnp.float32),
        ],
        compiler_params=pltpu.CompilerParams(
            needs_layout_passes=False, use_tc_tiling_on_sc=False),
    )
    return f(input_)


def kernel(input_):
    return _sparsemax_sc(input_)


# speedup vs baseline: 2.2295x; 2.0862x over previous
import jax
import jax.numpy as jnp
from jax import lax
from jax.experimental import pallas as pl
from jax.experimental.pallas import tpu as pltpu
from jax.experimental.pallas import tpu_sc as plsc

L = 16
NB = 256
ROWS = 128
N = 32768
VECS = N // L
NWORKERS = 32
ROWS_PER = ROWS // NWORKERS
W1 = 1.0 / NB
W2 = W1 / NB
U = 8
NBLK = VECS // U
CAP = 4096


def _splat(s, dtype=None):
    v = lax.broadcast(s, (L,))
    return v if dtype is None else v.astype(dtype)


def _sparsemax_body(in_hbm, out_hbm, row_a, row_b, cand_x, bmax, hcnt, hsum,
                    cnt_ref, si_a, si_b, so_a, so_b):
    c = lax.axis_index("c")
    s = lax.axis_index("s")
    wid = s * 2 + c

    lane = lax.iota(jnp.int32, L)
    ones = jnp.ones((L,), jnp.float32)
    zeros = jnp.zeros((L,), jnp.float32)
    izeros = jnp.zeros((L,), jnp.int32)
    cap_vec = jnp.full((L,), CAP, jnp.int32)

    def hist_level(row_v, top_vec, inv_w, w, use_cand, nv, tail):
        def zero_body(b, carry):
            for j in range(U):
                hcnt[b * U + j] = zeros
                hsum[b * U + j] = zeros
            return carry
        lax.fori_loop(0, NB // U, zero_body, 0)
        hcnt[NB] = zeros
        hsum[NB] = zeros

        inv_w_vec = jnp.full((L,), inv_w, jnp.float32)

        def scat_one(x, mask=None):
            tt = (top_vec - x) * inv_w_vec
            idx = jnp.clip(tt.astype(jnp.int32), 0, NB)
            plsc.addupdate_scatter(hcnt, [idx, lane], ones, mask=mask)
            plsc.addupdate_scatter(hsum, [idx, lane], x, mask=mask)

        @pl.when(use_cand)
        def _():
            def body(i, carry):
                scat_one(cand_x[pl.ds(pl.multiple_of(i * L, L), L)])
                return carry
            lax.fori_loop(0, nv, body, 0)
            xt = cand_x[pl.ds(pl.multiple_of(nv * L, L), L)]
            scat_one(xt, mask=lane < _splat(tail))

        @pl.when(jnp.logical_not(use_cand))
        def _():
            def body(i, carry):
                for j in range(U):
                    scat_one(row_v[pl.ds(pl.multiple_of((i * U + j) * L, L),
                                         L)])
                return carry
            lax.fori_loop(0, VECS // U, body, 0)

        def cum_body(b, carry):
            cc, cs = carry
            for j in range(U):
                cc = cc + hcnt[b * U + j]
                cs = cs + hsum[b * U + j]
                hcnt[b * U + j] = cc
                hsum[b * U + j] = cs
            return (cc, cs)
        lax.fori_loop(0, NB // U, cum_body, (zeros, zeros))

        w_vec = jnp.full((L,), w, jnp.float32)

        def g_nonneg(b):
            cvec = _splat(jnp.sum(hcnt[b]))
            svec = _splat(jnp.sum(hsum[b]))
            bf = _splat(b + 1).astype(jnp.float32)
            beta = top_vec - bf * w_vec
            g = svec - beta * cvec - ones
            return jnp.any(g >= 0.0)

        def bs_body(it, lohi):
            lo, hi = lohi
            mid = (lo + hi) >> 1
            pred = g_nonneg(mid)
            lo2 = jnp.where(pred, lo, mid + 1)
            hi2 = jnp.where(pred, mid, hi)
            done = lo >= hi
            return (jnp.where(done, lo, lo2), jnp.where(done, hi, hi2))

        bstar, _ = lax.fori_loop(0, 8, bs_body,
                                 (jnp.int32(0), jnp.int32(NB - 1)))
        kvec = _splat(jnp.sum(hcnt[bstar]))
        svec = _splat(jnp.sum(hsum[bstar]))
        bf = _splat(bstar).astype(jnp.float32)
        new_top = top_vec - bf * w_vec
        return kvec, svec, new_top

    def row_compute(row_v):
        def maxblk_body(i, g):
            bm = row_v[pl.ds(pl.multiple_of(i * U * L, L), L)]
            for j in range(1, U):
                bm = jnp.maximum(
                    bm, row_v[pl.ds(pl.multiple_of((i * U + j) * L, L), L)])
            bmax[i] = bm
            return jnp.maximum(g, bm)
        g = lax.fori_loop(0, NBLK, maxblk_body,
                          jnp.full((L,), -jnp.inf, jnp.float32))
        m_vec = _splat(jnp.max(g))
        thresh = m_vec - ones

        cnt_ref[0] = jnp.int32(0)

        def comp_body(i, carry):
            bm = bmax[i]

            @pl.when(jnp.any(bm > thresh))
            def _():
                cur = _splat(cnt_ref[0])
                accv = izeros
                for j in range(U):
                    x = row_v[pl.ds(pl.multiple_of((i * U + j) * L, L), L)]
                    mask = x > thresh
                    pref = plsc.cumsum(mask.astype(jnp.int32))
                    dest = cur + accv + pref - 1
                    okm = jnp.logical_and(mask, dest < cap_vec)
                    plsc.store_scatter(cand_x, [dest], x, mask=okm)
                    accv = accv + plsc.all_reduce_population_count(mask)
                tot = cur + accv
                cnt_ref[0] = tot[0]
            return carry
        lax.fori_loop(0, NBLK, comp_body, 0)
        ncand = cnt_ref[0]

        use_cand = ncand <= CAP
        nv = lax.shift_right_logical(ncand, 4)
        tail = jnp.bitwise_and(ncand, 15)

        _, _, top2 = hist_level(row_v, m_vec, float(NB), W1,
                                use_cand, nv, tail)
        kvec, svec, _ = hist_level(row_v, top2, float(NB * NB), W2,
                                   use_cand, nv, tail)
        tau = (svec - ones) / kvec

        def out_body(i, carry):
            for j in range(U):
                sl = pl.ds(pl.multiple_of((i * U + j) * L, L), L)
                row_v[sl] = jnp.maximum(row_v[sl] - tau, 0.0)
            return carry
        lax.fori_loop(0, VECS // U, out_body, 0)

    bufs = [row_a, row_b]
    isems = [si_a, si_b]
    osems = [so_a, so_b]
    rows = [wid * ROWS_PER + r for r in range(ROWS_PER)]
    in_h = {0: pltpu.async_copy(in_hbm.at[rows[0]], bufs[0], isems[0])}
    out_h = {}
    for r in range(ROWS_PER):
        b = r % 2
        if r + 1 < ROWS_PER:
            nb = (r + 1) % 2
            if r - 1 >= 0:
                out_h[r - 1].wait()
            in_h[r + 1] = pltpu.async_copy(in_hbm.at[rows[r + 1]], bufs[nb],
                                           isems[nb])
        in_h[r].wait()
        row_compute(bufs[b])
        out_h[r] = pltpu.async_copy(bufs[b], out_hbm.at[rows[r]], osems[b])
    out_h[ROWS_PER - 2].wait()
    out_h[ROWS_PER - 1].wait()


@jax.jit
def _sparsemax_sc(input_):
    mesh = plsc.VectorSubcoreMesh(core_axis_name="c", subcore_axis_name="s",
                                  num_cores=2, num_subcores=16)
    f = pl.kernel(
        _sparsemax_body,
        out_type=jax.ShapeDtypeStruct((ROWS, N), jnp.float32),
        mesh=mesh,
        scratch_types=[
            pltpu.VMEM((N,), jnp.float32),
            pltpu.VMEM((N,), jnp.float32),
            pltpu.VMEM((CAP + L,), jnp.float32),
            pltpu.VMEM((NBLK, L), jnp.float32),
            pltpu.VMEM((NB + 1, L), jnp.float32),
            pltpu.VMEM((NB + 1, L), jnp.float32),
            pltpu.SMEM((1,), jnp.int32),
            pltpu.SemaphoreType.DMA,
            pltpu.SemaphoreType.DMA,
            pltpu.SemaphoreType.DMA,
            pltpu.SemaphoreType.DMA,
        ],
        compiler_params=pltpu.CompilerParams(
            needs_layout_passes=False, use_tc_tiling_on_sc=False),
    )
    return f(input_)


def kernel(input_):
    return _sparsemax_sc(input_)


# speedup vs baseline: 2.4956x; 1.1194x over previous
import jax
import jax.numpy as jnp
from jax import lax
from jax.experimental import pallas as pl
from jax.experimental.pallas import tpu as pltpu
from jax.experimental.pallas import tpu_sc as plsc

L = 16
NB = 256
ROWS = 128
N = 32768
VECS = N // L
NWORKERS = 32
ROWS_PER = ROWS // NWORKERS
W1 = 1.0 / NB
W2 = W1 / NB
U = 8
NBLK = VECS // U
CAP = 4096


def _splat(s, dtype=None):
    v = lax.broadcast(s, (L,))
    return v if dtype is None else v.astype(dtype)


def _sparsemax_body(in_hbm, out_hbm, row_a, row_b, cand_x, bmax, hcnt, hsum,
                    cnt_ref, si_a, si_b, so_a, so_b):
    c = lax.axis_index("c")
    s = lax.axis_index("s")
    wid = s * 2 + c

    lane = lax.iota(jnp.int32, L)
    ones = jnp.ones((L,), jnp.float32)
    zeros = jnp.zeros((L,), jnp.float32)
    izeros = jnp.zeros((L,), jnp.int32)
    cap_vec = jnp.full((L,), CAP, jnp.int32)

    def hist_level(row_v, top_vec, inv_w, w, use_cand, nv, tail):
        def zero_body(b, carry):
            for j in range(U):
                hcnt[b * U + j] = zeros
                hsum[b * U + j] = zeros
            return carry
        lax.fori_loop(0, NB // U, zero_body, 0)
        hcnt[NB] = zeros
        hsum[NB] = zeros

        inv_w_vec = jnp.full((L,), inv_w, jnp.float32)

        def scat_one(x, mask=None):
            tt = (top_vec - x) * inv_w_vec
            idx = jnp.clip(tt.astype(jnp.int32), 0, NB)
            plsc.addupdate_scatter(hcnt, [idx, lane], ones, mask=mask)
            plsc.addupdate_scatter(hsum, [idx, lane], x, mask=mask)

        @pl.when(use_cand)
        def _():
            def body(i, carry):
                scat_one(cand_x[pl.ds(pl.multiple_of(i * L, L), L)])
                return carry
            lax.fori_loop(0, nv, body, 0)
            xt = cand_x[pl.ds(pl.multiple_of(nv * L, L), L)]
            scat_one(xt, mask=lane < _splat(tail))

        @pl.when(jnp.logical_not(use_cand))
        def _():
            def body(i, carry):
                for j in range(U):
                    scat_one(row_v[pl.ds(pl.multiple_of((i * U + j) * L, L),
                                         L)])
                return carry
            lax.fori_loop(0, VECS // U, body, 0)

        def cum_body(b, carry):
            cc, cs = carry
            for j in range(U):
                cc = cc + hcnt[b * U + j]
                cs = cs + hsum[b * U + j]
                hcnt[b * U + j] = cc
                hsum[b * U + j] = cs
            return (cc, cs)
        lax.fori_loop(0, NB // U, cum_body, (zeros, zeros))

        w_vec = jnp.full((L,), w, jnp.float32)

        def g_nonneg(b):
            cvec = _splat(jnp.sum(hcnt[b]))
            svec = _splat(jnp.sum(hsum[b]))
            bf = _splat(b + 1).astype(jnp.float32)
            beta = top_vec - bf * w_vec
            g = svec - beta * cvec - ones
            return jnp.any(g >= 0.0)

        def bs_body(it, lohi):
            lo, hi = lohi
            mid = (lo + hi) >> 1
            pred = g_nonneg(mid)
            lo2 = jnp.where(pred, lo, mid + 1)
            hi2 = jnp.where(pred, mid, hi)
            done = lo >= hi
            return (jnp.where(done, lo, lo2), jnp.where(done, hi, hi2))

        bstar, _ = lax.fori_loop(0, 8, bs_body,
                                 (jnp.int32(0), jnp.int32(NB - 1)))
        kvec = _splat(jnp.sum(hcnt[bstar]))
        svec = _splat(jnp.sum(hsum[bstar]))
        bf = _splat(bstar).astype(jnp.float32)
        new_top = top_vec - bf * w_vec
        return kvec, svec, new_top

    def row_compute(row_v):
        def maxblk_body(i, g):
            bm = row_v[pl.ds(pl.multiple_of(i * U * L, L), L)]
            for j in range(1, U):
                bm = jnp.maximum(
                    bm, row_v[pl.ds(pl.multiple_of((i * U + j) * L, L), L)])
            bmax[i] = bm
            return jnp.maximum(g, bm)
        g = lax.fori_loop(0, NBLK, maxblk_body,
                          jnp.full((L,), -jnp.inf, jnp.float32))
        m_vec = _splat(jnp.max(g))
        thresh = m_vec - ones

        cnt_ref[0] = jnp.int32(0)

        def comp_sb(sb, carry):
            acc = izeros
            for jj in range(16):
                bm = bmax[sb * 16 + jj]
                pc = plsc.all_reduce_population_count(bm > thresh)
                acc = acc + jnp.where(
                    pc > 0, jnp.full((L,), 1 << jj, jnp.int32), izeros)
            smask = acc[0]

            @pl.when(smask != 0)
            def _():
                def bit_body(jj, carry2):
                    bi = sb * 16 + jj

                    @pl.when(
                        jnp.bitwise_and(
                            lax.shift_right_logical(smask, jj), 1) != 0)
                    def __():
                        cur = _splat(cnt_ref[0])
                        accv = izeros
                        for j in range(U):
                            x = row_v[pl.ds(
                                pl.multiple_of((bi * U + j) * L, L), L)]
                            mask = x > thresh
                            pref = plsc.cumsum(mask.astype(jnp.int32))
                            dest = cur + accv + pref - 1
                            okm = jnp.logical_and(mask, dest < cap_vec)
                            plsc.store_scatter(cand_x, [dest], x, mask=okm)
                            accv = accv + plsc.all_reduce_population_count(
                                mask)
                        tot = cur + accv
                        cnt_ref[0] = tot[0]
                    return carry2
                lax.fori_loop(0, 16, bit_body, 0)
            return carry
        lax.fori_loop(0, NBLK // 16, comp_sb, 0)
        ncand = cnt_ref[0]

        use_cand = ncand <= CAP
        nv = lax.shift_right_logical(ncand, 4)
        tail = jnp.bitwise_and(ncand, 15)

        _, _, top2 = hist_level(row_v, m_vec, float(NB), W1,
                                use_cand, nv, tail)
        kvec, svec, _ = hist_level(row_v, top2, float(NB * NB), W2,
                                   use_cand, nv, tail)
        tau = (svec - ones) / kvec

        def out_body(i, carry):
            for j in range(U):
                sl = pl.ds(pl.multiple_of((i * U + j) * L, L), L)
                row_v[sl] = jnp.maximum(row_v[sl] - tau, 0.0)
            return carry
        lax.fori_loop(0, VECS // U, out_body, 0)

    bufs = [row_a, row_b]
    isems = [si_a, si_b]
    osems = [so_a, so_b]
    rows = [wid * ROWS_PER + r for r in range(ROWS_PER)]
    in_h = {0: pltpu.async_copy(in_hbm.at[rows[0]], bufs[0], isems[0])}
    out_h = {}
    for r in range(ROWS_PER):
        b = r % 2
        if r + 1 < ROWS_PER:
            nb = (r + 1) % 2
            if r - 1 >= 0:
                out_h[r - 1].wait()
            in_h[r + 1] = pltpu.async_copy(in_hbm.at[rows[r + 1]], bufs[nb],
                                           isems[nb])
        in_h[r].wait()
        row_compute(bufs[b])
        out_h[r] = pltpu.async_copy(bufs[b], out_hbm.at[rows[r]], osems[b])
    out_h[ROWS_PER - 2].wait()
    out_h[ROWS_PER - 1].wait()


@jax.jit
def _sparsemax_sc(input_):
    mesh = plsc.VectorSubcoreMesh(core_axis_name="c", subcore_axis_name="s",
                                  num_cores=2, num_subcores=16)
    f = pl.kernel(
        _sparsemax_body,
        out_type=jax.ShapeDtypeStruct((ROWS, N), jnp.float32),
        mesh=mesh,
        scratch_types=[
            pltpu.VMEM((N,), jnp.float32),
            pltpu.VMEM((N,), jnp.float32),
            pltpu.VMEM((CAP + L,), jnp.float32),
            pltpu.VMEM((NBLK, L), jnp.float32),
            pltpu.VMEM((NB + 1, L), jnp.float32),
            pltpu.VMEM((NB + 1, L), jnp.float32),
            pltpu.SMEM((1,), jnp.int32),
            pltpu.SemaphoreType.DMA,
            pltpu.SemaphoreType.DMA,
            pltpu.SemaphoreType.DMA,
            pltpu.SemaphoreType.DMA,
        ],
        compiler_params=pltpu.CompilerParams(
            needs_layout_passes=False, use_tc_tiling_on_sc=False),
    )
    return f(input_)


def kernel(input_):
    return _sparsemax_sc(input_)


# speedup vs baseline: 2.7066x; 1.0846x over previous
import jax
import jax.numpy as jnp
from jax import lax
from jax.experimental import pallas as pl
from jax.experimental.pallas import tpu as pltpu
from jax.experimental.pallas import tpu_sc as plsc

L = 16
NB = 128
ROWS = 128
N = 32768
VECS = N // L
NWORKERS = 32
ROWS_PER = ROWS // NWORKERS
W1 = 1.0 / NB
W2 = W1 / NB
U = 8
NBLK = VECS // U
CAP = 4096


def _splat(s, dtype=None):
    v = lax.broadcast(s, (L,))
    return v if dtype is None else v.astype(dtype)


def _sparsemax_body(in_hbm, out_hbm, row_a, row_b, cand_x, bmax, hcnt, hsum,
                    cnt_ref, si_a, si_b, so_a, so_b):
    c = lax.axis_index("c")
    s = lax.axis_index("s")
    wid = s * 2 + c

    lane = lax.iota(jnp.int32, L)
    ones = jnp.ones((L,), jnp.float32)
    zeros = jnp.zeros((L,), jnp.float32)
    izeros = jnp.zeros((L,), jnp.int32)
    cap_vec = jnp.full((L,), CAP, jnp.int32)

    def hist_level(row_v, top_vec, inv_w, w, use_cand, nv, tail):
        def zero_body(b, carry):
            for j in range(U):
                hcnt[b * U + j] = zeros
                hsum[b * U + j] = zeros
            return carry
        lax.fori_loop(0, NB // U, zero_body, 0)
        hcnt[NB] = zeros
        hsum[NB] = zeros

        inv_w_vec = jnp.full((L,), inv_w, jnp.float32)

        def scat_one(x, mask=None):
            tt = (top_vec - x) * inv_w_vec
            idx = jnp.clip(tt.astype(jnp.int32), 0, NB)
            plsc.addupdate_scatter(hcnt, [idx, lane], ones, mask=mask)
            plsc.addupdate_scatter(hsum, [idx, lane], x, mask=mask)

        @pl.when(use_cand)
        def _():
            def body(i, carry):
                scat_one(cand_x[pl.ds(pl.multiple_of(i * L, L), L)])
                return carry
            lax.fori_loop(0, nv, body, 0)
            xt = cand_x[pl.ds(pl.multiple_of(nv * L, L), L)]
            scat_one(xt, mask=lane < _splat(tail))

        @pl.when(jnp.logical_not(use_cand))
        def _():
            def body(i, carry):
                for j in range(U):
                    scat_one(row_v[pl.ds(pl.multiple_of((i * U + j) * L, L),
                                         L)])
                return carry
            lax.fori_loop(0, VECS // U, body, 0)

        def cum_body(b, carry):
            cc, cs = carry
            for j in range(U):
                cc = cc + hcnt[b * U + j]
                cs = cs + hsum[b * U + j]
                hcnt[b * U + j] = cc
                hsum[b * U + j] = cs
            return (cc, cs)
        lax.fori_loop(0, NB // U, cum_body, (zeros, zeros))

        w_vec = jnp.full((L,), w, jnp.float32)

        def g_nonneg(b):
            cvec = _splat(jnp.sum(hcnt[b]))
            svec = _splat(jnp.sum(hsum[b]))
            bf = _splat(b + 1).astype(jnp.float32)
            beta = top_vec - bf * w_vec
            g = svec - beta * cvec - ones
            return jnp.any(g >= 0.0)

        def bs_body(it, lohi):
            lo, hi = lohi
            mid = (lo + hi) >> 1
            pred = g_nonneg(mid)
            lo2 = jnp.where(pred, lo, mid + 1)
            hi2 = jnp.where(pred, mid, hi)
            done = lo >= hi
            return (jnp.where(done, lo, lo2), jnp.where(done, hi, hi2))

        bstar, _ = lax.fori_loop(0, 7, bs_body,
                                 (jnp.int32(0), jnp.int32(NB - 1)))
        kvec = _splat(jnp.sum(hcnt[bstar]))
        svec = _splat(jnp.sum(hsum[bstar]))
        bf = _splat(bstar).astype(jnp.float32)
        new_top = top_vec - bf * w_vec
        return kvec, svec, new_top

    def row_compute(row_v):
        def maxblk_body(i, g):
            bm = row_v[pl.ds(pl.multiple_of(i * U * L, L), L)]
            for j in range(1, U):
                bm = jnp.maximum(
                    bm, row_v[pl.ds(pl.multiple_of((i * U + j) * L, L), L)])
            bmax[i] = bm
            return jnp.maximum(g, bm)
        g = lax.fori_loop(0, NBLK, maxblk_body,
                          jnp.full((L,), -jnp.inf, jnp.float32))
        m_vec = _splat(jnp.max(g))
        thresh = m_vec - ones

        cnt_ref[0] = jnp.int32(0)

        def comp_sb(sb, carry):
            acc = izeros
            for jj in range(16):
                bm = bmax[sb * 16 + jj]
                pc = plsc.all_reduce_population_count(bm > thresh)
                acc = acc + jnp.where(
                    pc > 0, jnp.full((L,), 1 << jj, jnp.int32), izeros)
            smask = acc[0]

            def any_left(sm):
                return sm != 0

            def next_bit(sm):
                low = jnp.bitwise_and(sm, -sm)
                fbits = lax.bitcast_convert_type(
                    low.astype(jnp.float32), jnp.int32)
                jj = lax.shift_right_logical(fbits, 23) - 127
                bi = sb * 16 + jj
                cur = _splat(cnt_ref[0])
                accv = izeros
                for j in range(U):
                    x = row_v[pl.ds(
                        pl.multiple_of((bi * U + j) * L, L), L)]
                    mask = x > thresh
                    pref = plsc.cumsum(mask.astype(jnp.int32))
                    dest = cur + accv + pref - 1
                    okm = jnp.logical_and(mask, dest < cap_vec)
                    plsc.store_scatter(cand_x, [dest], x, mask=okm)
                    accv = accv + plsc.all_reduce_population_count(mask)
                tot = cur + accv
                cnt_ref[0] = tot[0]
                return sm - low

            lax.while_loop(any_left, next_bit, smask)
            return carry
        lax.fori_loop(0, NBLK // 16, comp_sb, 0)
        ncand = cnt_ref[0]

        use_cand = ncand <= CAP
        nv = lax.shift_right_logical(ncand, 4)
        tail = jnp.bitwise_and(ncand, 15)

        _, _, top2 = hist_level(row_v, m_vec, float(NB), W1,
                                use_cand, nv, tail)
        kvec, svec, _ = hist_level(row_v, top2, float(NB * NB), W2,
                                   use_cand, nv, tail)
        tau = (svec - ones) / kvec

        def out_body(i, carry):
            for j in range(U):
                sl = pl.ds(pl.multiple_of((i * U + j) * L, L), L)
                row_v[sl] = jnp.maximum(row_v[sl] - tau, 0.0)
            return carry
        lax.fori_loop(0, VECS // U, out_body, 0)

    bufs = [row_a, row_b]
    isems = [si_a, si_b]
    osems = [so_a, so_b]
    rows = [wid * ROWS_PER + r for r in range(ROWS_PER)]
    in_h = {0: pltpu.async_copy(in_hbm.at[rows[0]], bufs[0], isems[0])}
    out_h = {}
    for r in range(ROWS_PER):
        b = r % 2
        if r + 1 < ROWS_PER:
            nb = (r + 1) % 2
            if r - 1 >= 0:
                out_h[r - 1].wait()
            in_h[r + 1] = pltpu.async_copy(in_hbm.at[rows[r + 1]], bufs[nb],
                                           isems[nb])
        in_h[r].wait()
        row_compute(bufs[b])
        out_h[r] = pltpu.async_copy(bufs[b], out_hbm.at[rows[r]], osems[b])
    out_h[ROWS_PER - 2].wait()
    out_h[ROWS_PER - 1].wait()


@jax.jit
def _sparsemax_sc(input_):
    mesh = plsc.VectorSubcoreMesh(core_axis_name="c", subcore_axis_name="s",
                                  num_cores=2, num_subcores=16)
    f = pl.kernel(
        _sparsemax_body,
        out_type=jax.ShapeDtypeStruct((ROWS, N), jnp.float32),
        mesh=mesh,
        scratch_types=[
            pltpu.VMEM((N,), jnp.float32),
            pltpu.VMEM((N,), jnp.float32),
            pltpu.VMEM((CAP + L,), jnp.float32),
            pltpu.VMEM((NBLK, L), jnp.float32),
            pltpu.VMEM((NB + 1, L), jnp.float32),
            pltpu.VMEM((NB + 1, L), jnp.float32),
            pltpu.SMEM((1,), jnp.int32),
            pltpu.SemaphoreType.DMA,
            pltpu.SemaphoreType.DMA,
            pltpu.SemaphoreType.DMA,
            pltpu.SemaphoreType.DMA,
        ],
        compiler_params=pltpu.CompilerParams(
            needs_layout_passes=False, use_tc_tiling_on_sc=False),
    )
    return f(input_)


def kernel(input_):
    return _sparsemax_sc(input_)
